# baseline (speedup 1.0000x reference)
import functools

import jax
import jax.numpy as jnp
from jax import lax
from jax.experimental import pallas as pl
from jax.experimental.pallas import tpu as pltpu
from jax.experimental.pallas import tpu_sc as plsc

_NUM_FIELDS = 26
_VOCAB = 100000
_EMB = 32
_BATCH = 1024
_SEQ = 50
_NUM_DIM = 16
_LANES = 16

_OUT_ROWS = _NUM_DIM + _NUM_FIELDS * _EMB
_SB = 4
_NBLK = 13
_QB = _BATCH // _LANES


def _build_sc_call():
    info = plsc.get_sparse_core_info()
    nc, ns = info.num_cores, info.num_subcores
    nw = nc * ns
    ppw = (_NUM_FIELDS * _EMB) // nw

    mesh = plsc.VectorSubcoreMesh(core_axis_name="c", subcore_axis_name="s")

    @functools.partial(
        pl.kernel,
        mesh=mesh,
        compiler_params=pltpu.CompilerParams(needs_layout_passes=False),
        out_type=jax.ShapeDtypeStruct((_SEQ, _OUT_ROWS, _BATCH), jnp.float32),
        scratch_types=[
            pltpu.VMEM((_VOCAB,), jnp.float32),
            pltpu.VMEM((_SB, _BATCH), jnp.int32),
            pltpu.VMEM((_SB, _BATCH), jnp.int32),
            pltpu.VMEM((_SB, _BATCH), jnp.float32),
            pltpu.VMEM((_SB, _BATCH), jnp.float32),
            pltpu.SemaphoreType.DMA,
            pltpu.SemaphoreType.DMA,
            pltpu.SemaphoreType.DMA,
            pltpu.SemaphoreType.DMA,
            pltpu.SemaphoreType.DMA,
        ],
    )
    def fn(tab, cat, num, out, rowb, cb0, cb1, ob0, ob1,
           rsem, cs0, cs1, os0, os1):
        wid = lax.axis_index("s") * nc + lax.axis_index("c")
        dyn0 = wid * 0
        catb = (cb0, cb1)
        outb = (ob0, ob1)
        csem = (cs0, cs1)
        osem = (os0, os1)

        def num_slab(m):
            s = m // 4
            k0 = (m % 4) * 4
            pltpu.sync_copy(num.at[s, pl.ds(k0, _SB)], ob0)
            pltpu.sync_copy(ob0, out.at[s, pl.ds(k0, _SB)])

        for t in range(6):
            num_slab(wid + 32 * t)

        @pl.when(wid < 8)
        def _():
            num_slab(wid + 192)

        def unit_body(k, carry):
            d = k
            e = wid
            c = _NUM_DIM + _EMB * d + e
            hrow = pltpu.async_copy(tab.at[d, e], rowb, rsem)
            hcat = {0: pltpu.async_copy(
                cat.at[d, pl.ds(dyn0, _SB)], catb[0], csem[0])}
            hout = {}
            hrow.wait()
            for j in range(_NBLK):
                b = j % 2
                rows = _SB if j < _NBLK - 1 else _SEQ - _SB * (_NBLK - 1)
                if j + 1 < _NBLK:
                    nrows = (_SB if j + 1 < _NBLK - 1
                             else _SEQ - _SB * (_NBLK - 1))
                    hcat[j + 1] = pltpu.async_copy(
                        cat.at[d, pl.ds(dyn0 + _SB * (j + 1), nrows)],
                        catb[1 - b].at[pl.ds(0, nrows)], csem[1 - b])
                hcat.pop(j).wait()
                if j - 2 in hout:
                    hout.pop(j - 2).wait()
                for si in range(rows):
                    @plsc.parallel_loop(0, _QB, unroll=8)
                    def _(q, si=si, b=b):
                        sl = pl.ds(q * _LANES, _LANES)
                        idx = catb[b][si, sl]
                        outb[b][si, sl] = plsc.load_gather(rowb, [idx])
                hout[j] = pltpu.async_copy(
                    outb[b].at[pl.ds(0, rows)],
                    out.at[pl.ds(dyn0 + _SB * j, rows), c], osem[b])
            hout.pop(_NBLK - 2).wait()
            hout.pop(_NBLK - 1).wait()
            return carry

        lax.fori_loop(0, ppw, unit_body, 0)

    return fn


def kernel(num, cat, tables):
    tab_t = jnp.transpose(tables, (0, 2, 1))
    cat_t = jnp.transpose(cat.astype(jnp.int32), (1, 2, 0))
    num_t = jnp.transpose(num, (2, 1, 0))
    out_t = _build_sc_call()(tab_t, cat_t, num_t)
    return jnp.transpose(out_t, (2, 1, 0))


# speedup vs baseline: 1.0762x; 1.0762x over previous
import functools

import jax
import jax.numpy as jnp
from jax import lax
from jax.experimental import pallas as pl
from jax.experimental.pallas import tpu as pltpu
from jax.experimental.pallas import tpu_sc as plsc

_NUM_FIELDS = 26
_VOCAB = 100000
_EMB = 32
_BATCH = 1024
_SEQ = 50
_NUM_DIM = 16
_LANES = 16

_OUT_ROWS = _NUM_DIM + _NUM_FIELDS * _EMB
_SB = 4
_NBLK = 13
_QB = _BATCH // _LANES


def _build_sc_call():
    info = plsc.get_sparse_core_info()
    nc, ns = info.num_cores, info.num_subcores
    nw = nc * ns
    ppw = (_NUM_FIELDS * _EMB) // nw

    mesh = plsc.VectorSubcoreMesh(core_axis_name="c", subcore_axis_name="s")

    @functools.partial(
        pl.kernel,
        mesh=mesh,
        compiler_params=pltpu.CompilerParams(needs_layout_passes=False),
        out_type=jax.ShapeDtypeStruct((_SEQ, _OUT_ROWS, _BATCH), jnp.float32),
        scratch_types=[
            pltpu.VMEM((_VOCAB,), jnp.float32),
            pltpu.VMEM((_SB, _BATCH), jnp.int32),
            pltpu.VMEM((_SB, _BATCH), jnp.int32),
            pltpu.VMEM((_SB, _BATCH), jnp.float32),
            pltpu.VMEM((_SB, _BATCH), jnp.float32),
            pltpu.SemaphoreType.DMA,
            pltpu.SemaphoreType.DMA,
            pltpu.SemaphoreType.DMA,
            pltpu.SemaphoreType.DMA,
            pltpu.SemaphoreType.DMA,
        ],
    )
    def fn(tab, cat, num, out, rowb, cb0, cb1, ob0, ob1,
           rsem, cs0, cs1, os0, os1):
        wid = lax.axis_index("s") * nc + lax.axis_index("c")
        dyn0 = wid * 0
        catb = (cb0, cb1)
        outb = (ob0, ob1)
        csem = (cs0, cs1)
        osem = (os0, os1)

        def num_slab(m):
            s = m // 4
            k0 = (m % 4) * 4
            pltpu.sync_copy(num.at[s, pl.ds(k0, _SB)], ob0)
            pltpu.sync_copy(ob0, out.at[s, pl.ds(k0, _SB)])

        for t in range(6):
            num_slab(wid + 32 * t)

        @pl.when(wid < 8)
        def _():
            num_slab(wid + 192)

        def unit_body(k, carry):
            p = wid * ppw + k
            d = p // _EMB
            e = p - d * _EMB
            c = _NUM_DIM + _EMB * d + e
            hrow = pltpu.async_copy(tab.at[d, e], rowb, rsem)
            hcat = {0: pltpu.async_copy(
                cat.at[d, pl.ds(dyn0, _SB)], catb[0], csem[0])}
            hout = {}
            hrow.wait()
            for j in range(_NBLK):
                b = j % 2
                rows = _SB if j < _NBLK - 1 else _SEQ - _SB * (_NBLK - 1)
                if j + 1 < _NBLK:
                    nrows = (_SB if j + 1 < _NBLK - 1
                             else _SEQ - _SB * (_NBLK - 1))
                    hcat[j + 1] = pltpu.async_copy(
                        cat.at[d, pl.ds(dyn0 + _SB * (j + 1), nrows)],
                        catb[1 - b].at[pl.ds(0, nrows)], csem[1 - b])
                hcat.pop(j).wait()
                if j - 2 in hout:
                    hout.pop(j - 2).wait()
                @plsc.parallel_loop(0, rows * _QB, unroll=8)
                def _(q, b=b):
                    si = lax.shift_right_logical(q, 6)
                    sl = pl.ds((q & (_QB - 1)) * _LANES, _LANES)
                    idx = catb[b][si, sl]
                    outb[b][si, sl] = plsc.load_gather(rowb, [idx])
                hout[j] = pltpu.async_copy(
                    outb[b].at[pl.ds(0, rows)],
                    out.at[pl.ds(dyn0 + _SB * j, rows), c], osem[b])
            hout.pop(_NBLK - 2).wait()
            hout.pop(_NBLK - 1).wait()
            return carry

        lax.fori_loop(0, ppw, unit_body, 0)

    return fn


def kernel(num, cat, tables):
    tab_t = jnp.transpose(tables, (0, 2, 1))
    cat_t = jnp.transpose(cat.astype(jnp.int32), (1, 2, 0))
    num_t = jnp.transpose(num, (2, 1, 0))
    out_t = _build_sc_call()(tab_t, cat_t, num_t)
    return jnp.transpose(out_t, (2, 1, 0))


# speedup vs baseline: 1.3283x; 1.2343x over previous
import functools

import jax
import jax.numpy as jnp
from jax import lax
from jax.experimental import pallas as pl
from jax.experimental.pallas import tpu as pltpu
from jax.experimental.pallas import tpu_sc as plsc

_NUM_FIELDS = 26
_VOCAB = 100000
_EMB = 32
_BATCH = 1024
_SEQ = 50
_NUM_DIM = 16
_LANES = 16

_OUT_ROWS = _NUM_DIM + _NUM_FIELDS * _EMB
_SB = 4
_NBLK = 13
_QB = _BATCH // _LANES


def _build_sc_call():
    info = plsc.get_sparse_core_info()
    nc, ns = info.num_cores, info.num_subcores
    nw = nc * ns
    ppw = (_NUM_FIELDS * _EMB) // nw

    mesh = plsc.VectorSubcoreMesh(core_axis_name="c", subcore_axis_name="s")

    @functools.partial(
        pl.kernel,
        mesh=mesh,
        compiler_params=pltpu.CompilerParams(needs_layout_passes=False),
        out_type=jax.ShapeDtypeStruct((_SEQ, _OUT_ROWS, _BATCH), jnp.float32),
        scratch_types=[
            pltpu.VMEM((_VOCAB,), jnp.float32),
            pltpu.VMEM((_SB, _BATCH), jnp.int32),
            pltpu.VMEM((_SB, _BATCH), jnp.int32),
            pltpu.VMEM((_SB, _BATCH), jnp.int32),
            pltpu.VMEM((_SB, _BATCH), jnp.int32),
            pltpu.VMEM((_SB, _BATCH), jnp.float32),
            pltpu.VMEM((_SB, _BATCH), jnp.float32),
            pltpu.SemaphoreType.DMA,
            pltpu.SemaphoreType.DMA,
            pltpu.SemaphoreType.DMA,
            pltpu.SemaphoreType.DMA,
            pltpu.SemaphoreType.DMA,
            pltpu.SemaphoreType.DMA,
            pltpu.SemaphoreType.DMA,
        ],
    )
    def fn(tab, cat, num, out, rowb, cb0, cb1, cb2, cb3, ob0, ob1,
           rsem, cs0, cs1, cs2, cs3, os0, os1):
        wid = lax.axis_index("s") * nc + lax.axis_index("c")
        dyn0 = wid * 0
        catb = (cb0, cb1, cb2, cb3)
        outb = (ob0, ob1)
        csem = (cs0, cs1, cs2, cs3)
        osem = (os0, os1)

        def num_slab(m):
            s = m // 4
            k0 = (m % 4) * 4
            pltpu.sync_copy(num.at[s, pl.ds(k0, _SB)], ob0)
            pltpu.sync_copy(ob0, out.at[s, pl.ds(k0, _SB)])

        for t in range(6):
            num_slab(wid + 32 * t)

        @pl.when(wid < 8)
        def _():
            num_slab(wid + 192)

        def unit_body(k, carry):
            p = wid * ppw + k
            d = p // _EMB
            e = p - d * _EMB
            c = _NUM_DIM + _EMB * d + e
            hrow = pltpu.async_copy(tab.at[d, e], rowb, rsem)

            def fire_cat(j):
                nrows = _SB if j < _NBLK - 1 else _SEQ - _SB * (_NBLK - 1)
                return pltpu.async_copy(
                    cat.at[d, pl.ds(dyn0 + _SB * j, nrows)],
                    catb[j % 4].at[pl.ds(0, nrows)], csem[j % 4])

            hcat = {j: fire_cat(j) for j in range(3)}
            hout = {}
            hrow.wait()
            for j in range(_NBLK):
                b = j % 2
                cbx = j % 4
                rows = _SB if j < _NBLK - 1 else _SEQ - _SB * (_NBLK - 1)
                if j + 3 < _NBLK:
                    hcat[j + 3] = fire_cat(j + 3)
                hcat.pop(j).wait()
                if j - 2 in hout:
                    hout.pop(j - 2).wait()
                @plsc.parallel_loop(0, rows * _QB, unroll=8)
                def _(q, b=b, cbx=cbx):
                    si = lax.shift_right_logical(q, 6)
                    sl = pl.ds((q & (_QB - 1)) * _LANES, _LANES)
                    idx = catb[cbx][si, sl]
                    outb[b][si, sl] = plsc.load_gather(rowb, [idx])
                hout[j] = pltpu.async_copy(
                    outb[b].at[pl.ds(0, rows)],
                    out.at[pl.ds(dyn0 + _SB * j, rows), c], osem[b])
            hout.pop(_NBLK - 2).wait()
            hout.pop(_NBLK - 1).wait()
            return carry

        lax.fori_loop(0, ppw, unit_body, 0)

    return fn


def kernel(num, cat, tables):
    tab_t = jnp.transpose(tables, (0, 2, 1))
    cat_t = jnp.transpose(cat.astype(jnp.int32), (1, 2, 0))
    num_t = jnp.transpose(num, (2, 1, 0))
    out_t = _build_sc_call()(tab_t, cat_t, num_t)
    return jnp.transpose(out_t, (2, 1, 0))


# speedup vs baseline: 1.3292x; 1.0007x over previous
import functools

import jax
import jax.numpy as jnp
from jax import lax
from jax.experimental import pallas as pl
from jax.experimental.pallas import tpu as pltpu
from jax.experimental.pallas import tpu_sc as plsc

_NUM_FIELDS = 26
_VOCAB = 100000
_EMB = 32
_BATCH = 1024
_SEQ = 50
_NUM_DIM = 16
_LANES = 16

_OUT_ROWS = _NUM_DIM + _NUM_FIELDS * _EMB
_SB = 4
_NBLK = 13
_QB = _BATCH // _LANES


def _build_sc_call():
    info = plsc.get_sparse_core_info()
    nc, ns = info.num_cores, info.num_subcores
    nw = nc * ns
    ppw = (_NUM_FIELDS * _EMB) // nw

    mesh = plsc.VectorSubcoreMesh(core_axis_name="c", subcore_axis_name="s")

    @functools.partial(
        pl.kernel,
        mesh=mesh,
        compiler_params=pltpu.CompilerParams(needs_layout_passes=False),
        out_type=jax.ShapeDtypeStruct((_SEQ, _OUT_ROWS, _BATCH), jnp.float32),
        scratch_types=[
            pltpu.VMEM((_VOCAB,), jnp.float32),
            pltpu.VMEM((_SB, _BATCH), jnp.int32),
            pltpu.VMEM((_SB, _BATCH), jnp.int32),
            pltpu.VMEM((_SB, _BATCH), jnp.int32),
            pltpu.VMEM((_SB, _BATCH), jnp.int32),
            pltpu.VMEM((_SB, _BATCH), jnp.float32),
            pltpu.VMEM((_SB, _BATCH), jnp.float32),
            pltpu.VMEM((_SB, _BATCH), jnp.float32),
            pltpu.SemaphoreType.DMA,
            pltpu.SemaphoreType.DMA,
            pltpu.SemaphoreType.DMA,
            pltpu.SemaphoreType.DMA,
            pltpu.SemaphoreType.DMA,
            pltpu.SemaphoreType.DMA,
            pltpu.SemaphoreType.DMA,
            pltpu.SemaphoreType.DMA,
        ],
    )
    def fn(tab, cat, num, out, rowb, cb0, cb1, cb2, cb3, ob0, ob1, ob2,
           rsem, cs0, cs1, cs2, cs3, os0, os1, os2):
        wid = lax.axis_index("s") * nc + lax.axis_index("c")
        dyn0 = wid * 0
        catb = (cb0, cb1, cb2, cb3)
        outb = (ob0, ob1, ob2)
        csem = (cs0, cs1, cs2, cs3)
        osem = (os0, os1, os2)

        def num_slab(m):
            s = m // 4
            k0 = (m % 4) * 4
            pltpu.sync_copy(num.at[s, pl.ds(k0, _SB)], ob0)
            pltpu.sync_copy(ob0, out.at[s, pl.ds(k0, _SB)])

        for t in range(6):
            num_slab(wid + 32 * t)

        @pl.when(wid < 8)
        def _():
            num_slab(wid + 192)

        def unit_body(k, carry):
            p = wid * ppw + k
            d = p // _EMB
            e = p - d * _EMB
            c = _NUM_DIM + _EMB * d + e
            hrow = pltpu.async_copy(tab.at[d, e], rowb, rsem)

            def fire_cat(j):
                nrows = _SB if j < _NBLK - 1 else _SEQ - _SB * (_NBLK - 1)
                return pltpu.async_copy(
                    cat.at[d, pl.ds(dyn0 + _SB * j, nrows)],
                    catb[j % 4].at[pl.ds(0, nrows)], csem[j % 4])

            hcat = {j: fire_cat(j) for j in range(3)}
            hout = {}
            hrow.wait()
            for j in range(_NBLK):
                b = j % 3
                cbx = j % 4
                rows = _SB if j < _NBLK - 1 else _SEQ - _SB * (_NBLK - 1)
                if j + 3 < _NBLK:
                    hcat[j + 3] = fire_cat(j + 3)
                hcat.pop(j).wait()
                if j - 3 in hout:
                    hout.pop(j - 3).wait()
                @plsc.parallel_loop(0, rows * _QB, unroll=8)
                def _(q, b=b, cbx=cbx):
                    si = lax.shift_right_logical(q, 6)
                    sl = pl.ds((q & (_QB - 1)) * _LANES, _LANES)
                    idx = catb[cbx][si, sl]
                    outb[b][si, sl] = plsc.load_gather(rowb, [idx])
                hout[j] = pltpu.async_copy(
                    outb[b].at[pl.ds(0, rows)],
                    out.at[pl.ds(dyn0 + _SB * j, rows), c], osem[b])
            hout.pop(_NBLK - 3).wait()
            hout.pop(_NBLK - 2).wait()
            hout.pop(_NBLK - 1).wait()
            return carry

        lax.fori_loop(0, ppw, unit_body, 0)

    return fn


def kernel(num, cat, tables):
    tab_t = jnp.transpose(tables, (0, 2, 1))
    cat_t = jnp.transpose(cat.astype(jnp.int32), (1, 2, 0))
    num_t = jnp.transpose(num, (2, 1, 0))
    out_t = _build_sc_call()(tab_t, cat_t, num_t)
    return jnp.transpose(out_t, (2, 1, 0))
